# baseline (speedup 1.0000x reference)
import functools

import jax
import jax.numpy as jnp
from jax import lax
from jax.experimental import pallas as pl
from jax.experimental.pallas import tpu as pltpu
from jax.experimental.pallas import tpu_sc as plsc

NUM_K = 8
D = 32
VOCAB = 1024
B = 4
H = 224
W = 224
HW = H * W
N = B * HW

NW = 32
PW = N // NW
CH = 112
NCHUNK = PW // CH
SEG = 896
NROW = N // 4
BLKP = 4 * SEG
NS = HW // BLKP


def _filter_body(x_ref, idx_ref):
    w25 = jnp.float32(0.04)
    scale = jnp.float32(VOCAB - 1)
    k = pl.program_id(1)
    xq = x_ref[0, 0].astype(jnp.bfloat16).astype(jnp.float32)
    top = xq[:1, :]
    bot = xq[-1:, :]
    xv = jnp.concatenate([top, top, xq, bot, bot], axis=0)
    left = xv[:, :1]
    right = xv[:, -1:]
    xp = jnp.concatenate([left, left, xv, right, right], axis=1)
    cols = [xp[:, dx:dx + W] for dx in range(5)]
    acc = None
    for dy in range(5):
        for dx in range(5):
            p = cols[dx][dy:dy + H, :] * w25
            acc = p if acc is None else acc + p
    idx = (acc * scale).astype(jnp.int32) + k * VOCAB
    idx_ref[0, 0] = idx


def _compute_idx(x):
    return pl.pallas_call(
        _filter_body,
        grid=(B, NUM_K),
        in_specs=[pl.BlockSpec((1, 1, H, W), lambda b, k: (b, k, 0, 0))],
        out_specs=pl.BlockSpec((1, 1, H, W), lambda b, k: (k, b, 0, 0)),
        out_shape=jax.ShapeDtypeStruct((NUM_K, B, H, W), jnp.int32),
    )(x)


def _gather_sum_body(tab_hbm, idx_hbm, out_hbm, idx_v, rows_v, out_v, tab_sp,
                     gsem0, gsem1, osem0, osem1):
    wid = lax.axis_index("s") * 2 + lax.axis_index("c")
    gsem = (gsem0, gsem1)
    osem = (osem0, osem1)

    @pl.when(lax.axis_index("s") == 0)
    def _():
        pltpu.sync_copy(tab_hbm, tab_sp)

    pltpu.sync_copy(idx_hbm.at[:, pl.ds(wid * NCHUNK, NCHUNK), :], idx_v)
    plsc.subcore_barrier()

    def dst_slice(c):
        segglobal = wid * 7 + (c >> 3)
        j2 = jnp.bitwise_and(segglobal, 3)
        t = segglobal >> 2
        gr = t * SEG + jnp.bitwise_and(c, 7) * CH
        return out_hbm.at[pl.ds(gr, CH), pl.ds(j2 * D, D)]

    def fetch(c, par):
        for k in range(NUM_K):
            pltpu.make_async_copy(
                tab_sp.at[idx_v.at[k, c]], rows_v.at[par, k], gsem[par]
            ).start()

    def process(c, par):
        for k in range(NUM_K):
            pltpu.make_async_copy(
                tab_sp.at[idx_v.at[k, c]], rows_v.at[par, k], gsem[par]
            ).wait()

        @pl.when(c >= 2)
        def _():
            pltpu.make_async_copy(out_v.at[par], dst_slice(c), osem[par]).wait()

        def sum_body(i, carry):
            for j in range(D // 16):
                s = pl.ds(j * 16, 16)
                acc = rows_v[par, 0, i, s]
                for k in range(1, NUM_K):
                    acc = acc + rows_v[par, k, i, s]
                out_v[par, i, s] = acc
            return carry

        lax.fori_loop(0, CH, sum_body, 0, unroll=2)
        pltpu.make_async_copy(out_v.at[par], dst_slice(c), osem[par]).start()

    fetch(0, 0)

    def loop_body(i, carry):
        for par in range(2):
            c = 2 * i + par

            @pl.when(c + 1 < NCHUNK)
            def _():
                fetch(c + 1, 1 - par)

            process(c, par)
        return carry

    lax.fori_loop(0, NCHUNK // 2, loop_body, 0)

    for par in range(2):
        c = NCHUNK - 2 + par
        pltpu.make_async_copy(out_v.at[par], dst_slice(c), osem[par]).wait()


@functools.cache
def _gather_sum():
    mesh = plsc.VectorSubcoreMesh(core_axis_name="c", subcore_axis_name="s")
    return pl.kernel(
        _gather_sum_body,
        out_type=jax.ShapeDtypeStruct((NROW, 4 * D), jnp.float32),
        mesh=mesh,
        compiler_params=pltpu.CompilerParams(use_tc_tiling_on_sc=False),
        scratch_types=[
            pltpu.VMEM((NUM_K, NCHUNK, CH), jnp.int32),
            pltpu.VMEM((2, NUM_K, CH, D), jnp.float32),
            pltpu.VMEM((2, CH, D), jnp.float32),
            pltpu.VMEM_SHARED((NUM_K * VOCAB, D), jnp.float32),
            pltpu.SemaphoreType.DMA,
            pltpu.SemaphoreType.DMA,
            pltpu.SemaphoreType.DMA,
            pltpu.SemaphoreType.DMA,
        ],
    )


def _transpose_tanh_body(rows_ref, out_ref):
    a = rows_ref[...]
    eye = (lax.broadcasted_iota(jnp.int32, (128, 128), 0)
           == lax.broadcasted_iota(jnp.int32, (128, 128), 1)).astype(jnp.float32)
    at = lax.dot_general(eye, a, (((1,), (1,)), ((), ())),
                         preferred_element_type=jnp.float32)
    for j in range(4):
        out_ref[0, :, pl.ds(j * SEG, SEG)] = jnp.tanh(at[j * D:(j + 1) * D, :])


def _transpose_tanh(rows):
    return pl.pallas_call(
        _transpose_tanh_body,
        grid=(B, NS),
        in_specs=[pl.BlockSpec((SEG, 128), lambda b, s: (b * NS + s, 0))],
        out_specs=pl.BlockSpec((1, D, BLKP), lambda b, s: (b, 0, s)),
        out_shape=jax.ShapeDtypeStruct((B, D, HW), jnp.float32),
    )(rows)


def kernel(x, tables):
    idx = _compute_idx(x).reshape(NUM_K, NW * NCHUNK, CH)
    tab_flat = tables.reshape(NUM_K * VOCAB, D)
    rows = _gather_sum()(tab_flat, idx)
    out = _transpose_tanh(rows)
    return out.reshape(B, D, H, W)


# speedup vs baseline: 1.0708x; 1.0708x over previous
import functools

import jax
import jax.numpy as jnp
from jax import lax
from jax.experimental import pallas as pl
from jax.experimental.pallas import tpu as pltpu
from jax.experimental.pallas import tpu_sc as plsc

NUM_K = 8
D = 32
VOCAB = 1024
B = 4
H = 224
W = 224
HW = H * W
N = B * HW

NW = 32
PW = N // NW
CH = 112
NCHUNK = PW // CH
SEG = 896
NROW = N // 4
BLKP = 4 * SEG
NS = HW // BLKP


def _filter_body(x_ref, idx_ref):
    w25 = jnp.float32(0.04)
    scale = jnp.float32(VOCAB - 1)
    k = pl.program_id(1)
    xq = x_ref[0, 0].astype(jnp.bfloat16).astype(jnp.float32)
    top = xq[:1, :]
    bot = xq[-1:, :]
    xv = jnp.concatenate([top, top, xq, bot, bot], axis=0)
    left = xv[:, :1]
    right = xv[:, -1:]
    xp = jnp.concatenate([left, left, xv, right, right], axis=1)
    cols = [xp[:, dx:dx + W] for dx in range(5)]
    acc = None
    for dy in range(5):
        for dx in range(5):
            p = cols[dx][dy:dy + H, :] * w25
            acc = p if acc is None else acc + p
    idx = (acc * scale).astype(jnp.int32) + k * VOCAB
    idx_ref[0, 0] = idx


def _compute_idx(x):
    return pl.pallas_call(
        _filter_body,
        grid=(B, NUM_K),
        in_specs=[pl.BlockSpec((1, 1, H, W), lambda b, k: (b, k, 0, 0))],
        out_specs=pl.BlockSpec((1, 1, H, W), lambda b, k: (k, b, 0, 0)),
        out_shape=jax.ShapeDtypeStruct((NUM_K, B, H, W), jnp.int32),
    )(x)


def _gather_sum_body(tab_hbm, idx_hbm, out_hbm, idx_v, rows_v, out_v, tab_sp,
                     gsem0, gsem1, osem0, osem1):
    wid = lax.axis_index("s") * 2 + lax.axis_index("c")
    gsem = (gsem0, gsem1)
    osem = (osem0, osem1)

    @pl.when(lax.axis_index("s") == 0)
    def _():
        pltpu.sync_copy(tab_hbm, tab_sp)

    pltpu.sync_copy(idx_hbm.at[:, pl.ds(wid * NCHUNK, NCHUNK), :], idx_v)
    plsc.subcore_barrier()

    def dst_slice(c):
        segglobal = wid * 7 + (c >> 3)
        j2 = jnp.bitwise_and(segglobal, 3)
        t = segglobal >> 2
        gr = t * SEG + jnp.bitwise_and(c, 7) * CH
        return out_hbm.at[pl.ds(gr, CH), pl.ds(j2 * D, D)]

    def fetch(c, par):
        for k in range(NUM_K):
            pltpu.make_async_copy(
                tab_sp.at[idx_v.at[k, c]], rows_v.at[par, k], gsem[par]
            ).start()

    def process(c, par):
        for k in range(NUM_K):
            pltpu.make_async_copy(
                tab_sp.at[idx_v.at[k, c]], rows_v.at[par, k], gsem[par]
            ).wait()

        @pl.when(c >= 2)
        def _():
            pltpu.make_async_copy(out_v.at[par], dst_slice(c), osem[par]).wait()

        def sum_body(i, carry):
            for j in range(D // 16):
                s = pl.ds(j * 16, 16)
                acc = rows_v[par, 0, i, s]
                for k in range(1, NUM_K):
                    acc = acc + rows_v[par, k, i, s]
                out_v[par, i, s] = acc
            return carry

        lax.fori_loop(0, CH, sum_body, 0, unroll=2)
        pltpu.make_async_copy(out_v.at[par], dst_slice(c), osem[par]).start()

    fetch(0, 0)

    def loop_body(i, carry):
        for par in range(2):
            c = 2 * i + par

            @pl.when(c + 1 < NCHUNK)
            def _():
                fetch(c + 1, 1 - par)

            process(c, par)
        return carry

    lax.fori_loop(0, NCHUNK // 2, loop_body, 0)

    for par in range(2):
        c = NCHUNK - 2 + par
        pltpu.make_async_copy(out_v.at[par], dst_slice(c), osem[par]).wait()


@functools.cache
def _gather_sum():
    mesh = plsc.VectorSubcoreMesh(core_axis_name="c", subcore_axis_name="s")
    return pl.kernel(
        _gather_sum_body,
        out_type=jax.ShapeDtypeStruct((NROW, 4 * D), jnp.float32),
        mesh=mesh,
        compiler_params=pltpu.CompilerParams(use_tc_tiling_on_sc=False),
        scratch_types=[
            pltpu.VMEM((NUM_K, NCHUNK, CH), jnp.int32),
            pltpu.VMEM((2, NUM_K, CH, D), jnp.float32),
            pltpu.VMEM((2, CH, D), jnp.float32),
            pltpu.VMEM_SHARED((NUM_K * VOCAB, D), jnp.float32),
            pltpu.SemaphoreType.DMA,
            pltpu.SemaphoreType.DMA,
            pltpu.SemaphoreType.DMA,
            pltpu.SemaphoreType.DMA,
        ],
    )


HB = BLKP // W


def _transpose_tanh_body(rows_ref, out_ref):
    a = rows_ref[...]
    eye = (lax.broadcasted_iota(jnp.int32, (128, 128), 0)
           == lax.broadcasted_iota(jnp.int32, (128, 128), 1)).astype(jnp.float32)
    at = lax.dot_general(eye, a, (((1,), (1,)), ((), ())),
                         preferred_element_type=jnp.float32)
    for hh in range(HB):
        j2, r = divmod(hh, 4)
        piece = at[j2 * D:(j2 + 1) * D, r * W:(r + 1) * W]
        out_ref[0, :, hh, :] = jnp.tanh(piece)


def _transpose_tanh(rows):
    return pl.pallas_call(
        _transpose_tanh_body,
        grid=(B, NS),
        in_specs=[pl.BlockSpec((SEG, 128), lambda b, s: (b * NS + s, 0))],
        out_specs=pl.BlockSpec((1, D, HB, W), lambda b, s: (b, 0, s, 0)),
        out_shape=jax.ShapeDtypeStruct((B, D, H, W), jnp.float32),
    )(rows)


def kernel(x, tables):
    idx = _compute_idx(x).reshape(NUM_K, NW * NCHUNK, CH)
    tab_flat = tables.reshape(NUM_K * VOCAB, D)
    rows = _gather_sum()(tab_flat, idx)
    return _transpose_tanh(rows)


# speedup vs baseline: 1.4204x; 1.3264x over previous
import functools

import jax
import jax.numpy as jnp
from jax import lax
from jax.experimental import pallas as pl
from jax.experimental.pallas import tpu as pltpu
from jax.experimental.pallas import tpu_sc as plsc

NUM_K = 8
D = 32
VOCAB = 1024
B = 4
H = 224
W = 224
HW = H * W

NW = 32
PWQ = HW // NW
CH = 112
NCHUNK = PWQ // CH
NROWQ = HW // 4
NS = NROWQ // (4 * W)
HB = H // NS


def _filter_body(x_ref, idx_ref):
    w25 = jnp.float32(0.04)
    scale = jnp.float32(VOCAB - 1)
    k = pl.program_id(0)
    xq = x_ref[0, 0].astype(jnp.bfloat16).astype(jnp.float32)
    top = xq[:1, :]
    bot = xq[-1:, :]
    xv = jnp.concatenate([top, top, xq, bot, bot], axis=0)
    left = xv[:, :1]
    right = xv[:, -1:]
    xp = jnp.concatenate([left, left, xv, right, right], axis=1)
    cols = [xp[:, dx:dx + W] for dx in range(5)]
    acc = None
    for dy in range(5):
        for dx in range(5):
            p = cols[dx][dy:dy + H, :] * w25
            acc = p if acc is None else acc + p
    idx = (acc * scale).astype(jnp.int32) + k * VOCAB
    idx_ref[0, 0] = idx


def _compute_idx(x, b):
    return pl.pallas_call(
        _filter_body,
        grid=(NUM_K,),
        in_specs=[pl.BlockSpec((1, 1, H, W), lambda k: (b, k, 0, 0))],
        out_specs=pl.BlockSpec((1, 1, H, W), lambda k: (k, 0, 0, 0)),
        out_shape=jax.ShapeDtypeStruct((NUM_K, 1, H, W), jnp.int32),
    )(x)


def _gather_sum_body(tab_hbm, idx_hbm, out_hbm, idx_v, rows_v, out_v, tab_sp,
                     gsem0, gsem1, osem0, osem1):
    wid = lax.axis_index("s") * 2 + lax.axis_index("c")
    gsem = (gsem0, gsem1)
    osem = (osem0, osem1)

    @pl.when(lax.axis_index("s") == 0)
    def _():
        pltpu.sync_copy(tab_hbm, tab_sp)

    pltpu.sync_copy(idx_hbm.at[:, pl.ds(wid * NCHUNK, NCHUNK), :], idx_v)
    plsc.subcore_barrier()

    def dst_slice(c):
        segw = wid * 7 + (c >> 1)
        j2 = jnp.bitwise_and(segw, 3)
        gr = (segw >> 2) * W + jnp.bitwise_and(c, 1) * CH
        return out_hbm.at[pl.ds(gr, CH), pl.ds(j2 * D, D)]

    def fetch(c, par):
        for k in range(NUM_K):
            pltpu.make_async_copy(
                tab_sp.at[idx_v.at[k, c]], rows_v.at[par, k], gsem[par]
            ).start()

    def process(c, par):
        for k in range(NUM_K):
            pltpu.make_async_copy(
                tab_sp.at[idx_v.at[k, c]], rows_v.at[par, k], gsem[par]
            ).wait()

        @pl.when(c >= 2)
        def _():
            pltpu.make_async_copy(out_v.at[par], dst_slice(c), osem[par]).wait()

        def sum_body(i, carry):
            for j in range(D // 16):
                s = pl.ds(j * 16, 16)
                acc = rows_v[par, 0, i, s]
                for k in range(1, NUM_K):
                    acc = acc + rows_v[par, k, i, s]
                out_v[par, i, s] = acc
            return carry

        lax.fori_loop(0, CH, sum_body, 0, unroll=2)
        pltpu.make_async_copy(out_v.at[par], dst_slice(c), osem[par]).start()

    fetch(0, 0)

    def loop_body(i, carry):
        for par in range(2):
            c = 2 * i + par

            @pl.when(c + 1 < NCHUNK)
            def _():
                fetch(c + 1, 1 - par)

            process(c, par)
        return carry

    lax.fori_loop(0, NCHUNK // 2, loop_body, 0)

    for par in range(2):
        c = NCHUNK - 2 + par
        pltpu.make_async_copy(out_v.at[par], dst_slice(c), osem[par]).wait()


@functools.cache
def _gather_sum():
    mesh = plsc.VectorSubcoreMesh(core_axis_name="c", subcore_axis_name="s")
    return pl.kernel(
        _gather_sum_body,
        out_type=jax.ShapeDtypeStruct((NROWQ, 4 * D), jnp.float32),
        mesh=mesh,
        compiler_params=pltpu.CompilerParams(use_tc_tiling_on_sc=False),
        scratch_types=[
            pltpu.VMEM((NUM_K, NCHUNK, CH), jnp.int32),
            pltpu.VMEM((2, NUM_K, CH, D), jnp.float32),
            pltpu.VMEM((2, CH, D), jnp.float32),
            pltpu.VMEM_SHARED((NUM_K * VOCAB, D), jnp.float32),
            pltpu.SemaphoreType.DMA,
            pltpu.SemaphoreType.DMA,
            pltpu.SemaphoreType.DMA,
            pltpu.SemaphoreType.DMA,
        ],
    )


def _transpose_tanh_body(rows_ref, out_ref):
    a = rows_ref[...]
    eye = (lax.broadcasted_iota(jnp.int32, (128, 128), 0)
           == lax.broadcasted_iota(jnp.int32, (128, 128), 1)).astype(jnp.float32)
    at = lax.dot_general(eye, a, (((1,), (1,)), ((), ())),
                         preferred_element_type=jnp.float32)
    for hh in range(HB):
        bg, j2 = divmod(hh, 4)
        piece = at[j2 * D:(j2 + 1) * D, bg * W:(bg + 1) * W]
        out_ref[0, :, 4 * bg + j2, :] = jnp.tanh(piece)


def _transpose_tanh(rows, buf, b):
    if buf is None:
        in_specs = [pl.BlockSpec((4 * W, 128), lambda s: (s, 0))]
        args = (rows,)
        kwargs = {}
    else:
        in_specs = [
            pl.BlockSpec((4 * W, 128), lambda s: (s, 0)),
            pl.BlockSpec(memory_space=pl.ANY),
        ]
        args = (rows, buf)
        kwargs = {"input_output_aliases": {1: 0}}

    def body(rows_ref, *rest):
        _transpose_tanh_body(rows_ref, rest[-1])

    return pl.pallas_call(
        body,
        grid=(NS,),
        in_specs=in_specs,
        out_specs=pl.BlockSpec((1, D, HB, W), lambda s: (b, 0, s, 0)),
        out_shape=jax.ShapeDtypeStruct((B, D, H, W), jnp.float32),
        **kwargs,
    )(*args)


def kernel(x, tables):
    tab_flat = tables.reshape(NUM_K * VOCAB, D)
    idxs = [
        _compute_idx(x, b).reshape(NUM_K, NW * NCHUNK, CH) for b in range(B)
    ]
    rows = [_gather_sum()(tab_flat, idx) for idx in idxs]
    buf = None
    for b in range(B):
        buf = _transpose_tanh(rows[b], buf, b)
    return buf


# speedup vs baseline: 1.4586x; 1.0269x over previous
import functools

import jax
import jax.numpy as jnp
from jax import lax
from jax.experimental import pallas as pl
from jax.experimental.pallas import tpu as pltpu
from jax.experimental.pallas import tpu_sc as plsc

NUM_K = 8
D = 32
VOCAB = 1024
B = 4
H = 224
W = 224
HW = H * W

NW = 32
PWQ = HW // NW
CH = 112
NCHUNK = PWQ // CH
NROWQ = HW // 4
NS = NROWQ // (4 * W)
HB = H // NS


def _filter_body(x_ref, idx_ref, cols_ref):
    w25 = jnp.float32(0.04)
    scale = jnp.float32(VOCAB - 1)
    k = pl.program_id(0)
    xq = x_ref[0, 0].astype(jnp.bfloat16).astype(jnp.float32)
    top = xq[:1, :]
    bot = xq[-1:, :]
    xv = jnp.concatenate([top, top, xq, bot, bot], axis=0)
    left = xv[:, :1]
    right = xv[:, -1:]
    xp = jnp.concatenate([left, left, xv, right, right], axis=1)
    for dx in range(5):
        cols_ref[dx] = xp[:, dx:dx + W] * w25
    acc = None
    for dy in range(5):
        for dx in range(5):
            p = cols_ref[dx, dy:dy + H, :]
            acc = p if acc is None else acc + p
    idx = (acc * scale).astype(jnp.int32) + k * VOCAB
    idx_ref[0, 0] = idx


def _compute_idx(x, b):
    return pl.pallas_call(
        _filter_body,
        grid=(NUM_K,),
        in_specs=[pl.BlockSpec((1, 1, H, W), lambda k: (b, k, 0, 0))],
        out_specs=pl.BlockSpec((1, 1, H, W), lambda k: (k, 0, 0, 0)),
        out_shape=jax.ShapeDtypeStruct((NUM_K, 1, H, W), jnp.int32),
        scratch_shapes=[pltpu.VMEM((5, H + 4, W), jnp.float32)],
    )(x)


def _gather_sum_body(tab_hbm, idx_hbm, out_hbm, idx_v, rows_v, out_v, tab_sp,
                     gsem0, gsem1, osem0, osem1):
    wid = lax.axis_index("s") * 2 + lax.axis_index("c")
    gsem = (gsem0, gsem1)
    osem = (osem0, osem1)

    @pl.when(lax.axis_index("s") == 0)
    def _():
        pltpu.sync_copy(tab_hbm, tab_sp)

    pltpu.sync_copy(idx_hbm.at[:, pl.ds(wid * NCHUNK, NCHUNK), :], idx_v)
    plsc.subcore_barrier()

    def dst_slice(c):
        segw = wid * 7 + (c >> 1)
        j2 = jnp.bitwise_and(segw, 3)
        gr = (segw >> 2) * W + jnp.bitwise_and(c, 1) * CH
        return out_hbm.at[pl.ds(gr, CH), pl.ds(j2 * D, D)]

    def fetch(c, par):
        for k in range(NUM_K):
            pltpu.make_async_copy(
                tab_sp.at[idx_v.at[k, c]], rows_v.at[par, k], gsem[par]
            ).start()

    def process(c, par):
        for k in range(NUM_K):
            pltpu.make_async_copy(
                tab_sp.at[idx_v.at[k, c]], rows_v.at[par, k], gsem[par]
            ).wait()

        @pl.when(c >= 2)
        def _():
            pltpu.make_async_copy(out_v.at[par], dst_slice(c), osem[par]).wait()

        def sum_body(i, carry):
            for j in range(D // 16):
                s = pl.ds(j * 16, 16)
                acc = rows_v[par, 0, i, s]
                for k in range(1, NUM_K):
                    acc = acc + rows_v[par, k, i, s]
                out_v[par, i, s] = acc
            return carry

        lax.fori_loop(0, CH, sum_body, 0, unroll=2)
        pltpu.make_async_copy(out_v.at[par], dst_slice(c), osem[par]).start()

    fetch(0, 0)

    def loop_body(i, carry):
        for par in range(2):
            c = 2 * i + par

            @pl.when(c + 1 < NCHUNK)
            def _():
                fetch(c + 1, 1 - par)

            process(c, par)
        return carry

    lax.fori_loop(0, NCHUNK // 2, loop_body, 0)

    for par in range(2):
        c = NCHUNK - 2 + par
        pltpu.make_async_copy(out_v.at[par], dst_slice(c), osem[par]).wait()


@functools.cache
def _gather_sum():
    mesh = plsc.VectorSubcoreMesh(core_axis_name="c", subcore_axis_name="s")
    return pl.kernel(
        _gather_sum_body,
        out_type=jax.ShapeDtypeStruct((NROWQ, 4 * D), jnp.float32),
        mesh=mesh,
        compiler_params=pltpu.CompilerParams(use_tc_tiling_on_sc=False),
        scratch_types=[
            pltpu.VMEM((NUM_K, NCHUNK, CH), jnp.int32),
            pltpu.VMEM((2, NUM_K, CH, D), jnp.float32),
            pltpu.VMEM((2, CH, D), jnp.float32),
            pltpu.VMEM_SHARED((NUM_K * VOCAB, D), jnp.float32),
            pltpu.SemaphoreType.DMA,
            pltpu.SemaphoreType.DMA,
            pltpu.SemaphoreType.DMA,
            pltpu.SemaphoreType.DMA,
        ],
    )


def _transpose_tanh_body(rows_ref, out_ref):
    a = rows_ref[...]
    eye = (lax.broadcasted_iota(jnp.int32, (128, 128), 0)
           == lax.broadcasted_iota(jnp.int32, (128, 128), 1)).astype(jnp.float32)
    at = lax.dot_general(eye, a, (((1,), (1,)), ((), ())),
                         preferred_element_type=jnp.float32)
    for hh in range(HB):
        bg, j2 = divmod(hh, 4)
        piece = at[j2 * D:(j2 + 1) * D, bg * W:(bg + 1) * W]
        out_ref[0, :, 4 * bg + j2, :] = jnp.tanh(piece)


def _transpose_tanh(rows, buf, b):
    if buf is None:
        in_specs = [pl.BlockSpec((4 * W, 128), lambda s: (s, 0))]
        args = (rows,)
        kwargs = {}
    else:
        in_specs = [
            pl.BlockSpec((4 * W, 128), lambda s: (s, 0)),
            pl.BlockSpec(memory_space=pl.ANY),
        ]
        args = (rows, buf)
        kwargs = {"input_output_aliases": {1: 0}}

    def body(rows_ref, *rest):
        _transpose_tanh_body(rows_ref, rest[-1])

    return pl.pallas_call(
        body,
        grid=(NS,),
        in_specs=in_specs,
        out_specs=pl.BlockSpec((1, D, HB, W), lambda s: (b, 0, s, 0)),
        out_shape=jax.ShapeDtypeStruct((B, D, H, W), jnp.float32),
        **kwargs,
    )(*args)


def kernel(x, tables):
    tab_flat = tables.reshape(NUM_K * VOCAB, D)
    idxs = [
        _compute_idx(x, b).reshape(NUM_K, NW * NCHUNK, CH) for b in range(B)
    ]
    rows = [_gather_sum()(tab_flat, idx) for idx in idxs]
    buf = None
    for b in range(B):
        buf = _transpose_tanh(rows[b], buf, b)
    return buf


# speedup vs baseline: 1.5012x; 1.0292x over previous
import functools

import jax
import jax.numpy as jnp
from jax import lax
from jax.experimental import pallas as pl
from jax.experimental.pallas import tpu as pltpu
from jax.experimental.pallas import tpu_sc as plsc

NUM_K = 8
D = 32
VOCAB = 1024
B = 4
H = 224
W = 224
HW = H * W

NW = 32
PWQ = HW // NW
CH = 112
NCHUNK = PWQ // CH
NROWQ = HW // 4
NS = NROWQ // (4 * W)
HB = H // NS


def _filter_body(x_ref, idx_ref, cols_ref):
    w25 = jnp.float32(0.04)
    scale = jnp.float32(VOCAB - 1)
    k = pl.program_id(0)
    xq = x_ref[0, 0].astype(jnp.bfloat16).astype(jnp.float32)
    top = xq[:1, :]
    bot = xq[-1:, :]
    xv = jnp.concatenate([top, top, xq, bot, bot], axis=0)
    left = xv[:, :1]
    right = xv[:, -1:]
    xp = jnp.concatenate([left, left, xv, right, right], axis=1)
    for dx in range(5):
        cols_ref[dx] = xp[:, dx:dx + W] * w25
    acc = None
    for dy in range(5):
        for dx in range(5):
            p = cols_ref[dx, dy:dy + H, :]
            acc = p if acc is None else acc + p
    idx = (acc * scale).astype(jnp.int32) + k * VOCAB
    idx_ref[0, 0] = idx


def _compute_idx(x, b):
    return pl.pallas_call(
        _filter_body,
        grid=(NUM_K,),
        in_specs=[pl.BlockSpec((1, 1, H, W), lambda k: (b, k, 0, 0))],
        out_specs=pl.BlockSpec((1, 1, H, W), lambda k: (k, 0, 0, 0)),
        out_shape=jax.ShapeDtypeStruct((NUM_K, 1, H, W), jnp.int32),
        scratch_shapes=[pltpu.VMEM((5, H + 4, W), jnp.float32)],
    )(x)


def _gather_sum_body(tab_hbm, idx_hbm, out_hbm, idx_v, rows_v, out_v, tab_sp,
                     gsem0, gsem1, osem0, osem1):
    wid = lax.axis_index("s") * 2 + lax.axis_index("c")
    gsem = (gsem0, gsem1)
    osem = (osem0, osem1)

    sid = lax.axis_index("s")
    tchunk = NUM_K * VOCAB // 16
    tcp = pltpu.make_async_copy(
        tab_hbm.at[pl.ds(sid * tchunk, tchunk)],
        tab_sp.at[pl.ds(sid * tchunk, tchunk)], gsem0)
    icp = pltpu.make_async_copy(
        idx_hbm.at[:, pl.ds(wid * NCHUNK, NCHUNK), :], idx_v, gsem1)
    tcp.start()
    icp.start()
    tcp.wait()
    icp.wait()
    plsc.subcore_barrier()

    def dst_slice(c):
        segw = wid * 7 + (c >> 1)
        j2 = jnp.bitwise_and(segw, 3)
        gr = (segw >> 2) * W + jnp.bitwise_and(c, 1) * CH
        return out_hbm.at[pl.ds(gr, CH), pl.ds(j2 * D, D)]

    def fetch(c, par):
        for k in range(NUM_K):
            pltpu.make_async_copy(
                tab_sp.at[idx_v.at[k, c]], rows_v.at[par, k], gsem[par]
            ).start()

    def process(c, par):
        for k in range(NUM_K):
            pltpu.make_async_copy(
                tab_sp.at[idx_v.at[k, c]], rows_v.at[par, k], gsem[par]
            ).wait()

        @pl.when(c >= 2)
        def _():
            pltpu.make_async_copy(out_v.at[par], dst_slice(c), osem[par]).wait()

        def sum_body(i, carry):
            for j in range(D // 16):
                s = pl.ds(j * 16, 16)
                acc = rows_v[par, 0, i, s]
                for k in range(1, NUM_K):
                    acc = acc + rows_v[par, k, i, s]
                out_v[par, i, s] = acc
            return carry

        lax.fori_loop(0, CH, sum_body, 0, unroll=2)
        pltpu.make_async_copy(out_v.at[par], dst_slice(c), osem[par]).start()

    fetch(0, 0)

    def loop_body(i, carry):
        for par in range(2):
            c = 2 * i + par

            @pl.when(c + 1 < NCHUNK)
            def _():
                fetch(c + 1, 1 - par)

            process(c, par)
        return carry

    lax.fori_loop(0, NCHUNK // 2, loop_body, 0)

    for par in range(2):
        c = NCHUNK - 2 + par
        pltpu.make_async_copy(out_v.at[par], dst_slice(c), osem[par]).wait()


@functools.cache
def _gather_sum():
    mesh = plsc.VectorSubcoreMesh(core_axis_name="c", subcore_axis_name="s")
    return pl.kernel(
        _gather_sum_body,
        out_type=jax.ShapeDtypeStruct((NROWQ, 4 * D), jnp.float32),
        mesh=mesh,
        compiler_params=pltpu.CompilerParams(use_tc_tiling_on_sc=False),
        scratch_types=[
            pltpu.VMEM((NUM_K, NCHUNK, CH), jnp.int32),
            pltpu.VMEM((2, NUM_K, CH, D), jnp.float32),
            pltpu.VMEM((2, CH, D), jnp.float32),
            pltpu.VMEM_SHARED((NUM_K * VOCAB, D), jnp.float32),
            pltpu.SemaphoreType.DMA,
            pltpu.SemaphoreType.DMA,
            pltpu.SemaphoreType.DMA,
            pltpu.SemaphoreType.DMA,
        ],
    )


def _transpose_tanh_body(rows_ref, out_ref):
    a = rows_ref[...]
    eye = (lax.broadcasted_iota(jnp.int32, (128, 128), 0)
           == lax.broadcasted_iota(jnp.int32, (128, 128), 1)).astype(jnp.float32)
    at = lax.dot_general(eye, a, (((1,), (1,)), ((), ())),
                         preferred_element_type=jnp.float32)
    for hh in range(HB):
        bg, j2 = divmod(hh, 4)
        piece = at[j2 * D:(j2 + 1) * D, bg * W:(bg + 1) * W]
        out_ref[0, :, 4 * bg + j2, :] = jnp.tanh(piece)


def _transpose_tanh(rows, buf, b):
    if buf is None:
        in_specs = [pl.BlockSpec((4 * W, 128), lambda s: (s, 0))]
        args = (rows,)
        kwargs = {}
    else:
        in_specs = [
            pl.BlockSpec((4 * W, 128), lambda s: (s, 0)),
            pl.BlockSpec(memory_space=pl.ANY),
        ]
        args = (rows, buf)
        kwargs = {"input_output_aliases": {1: 0}}

    def body(rows_ref, *rest):
        _transpose_tanh_body(rows_ref, rest[-1])

    return pl.pallas_call(
        body,
        grid=(NS,),
        in_specs=in_specs,
        out_specs=pl.BlockSpec((1, D, HB, W), lambda s: (b, 0, s, 0)),
        out_shape=jax.ShapeDtypeStruct((B, D, H, W), jnp.float32),
        **kwargs,
    )(*args)


def kernel(x, tables):
    tab_flat = tables.reshape(NUM_K * VOCAB, D)
    idxs = [
        _compute_idx(x, b).reshape(NUM_K, NW * NCHUNK, CH) for b in range(B)
    ]
    rows = [_gather_sum()(tab_flat, idx) for idx in idxs]
    buf = None
    for b in range(B):
        buf = _transpose_tanh(rows[b], buf, b)
    return buf
